# Initial kernel scaffold; baseline (speedup 1.0000x reference)
import jax
import jax.numpy as jnp
from jax.experimental import pallas as pl


def kernel(embeddings, src_mask, pitch_target, energy_target, pitch_bins, energy_bins, pitch_emb, energy_emb, p_params, e_params):
    raise NotImplementedError("write your pallas kernel here")


# speedup vs baseline: 30.7457x; 30.7457x over previous
import jax
import jax.numpy as jnp
from jax.experimental import pallas as pl

B, T, H = 64, 2048, 256
NBINS, OUT, FILT, K = 256, 256, 256, 3
_EPS = 1e-5


def _conv3(x, w_ref, b_row):
    y0 = jnp.dot(x, w_ref[0], preferred_element_type=jnp.float32)
    y1 = jnp.dot(x, w_ref[1], preferred_element_type=jnp.float32)
    y2 = jnp.dot(x, w_ref[2], preferred_element_type=jnp.float32)
    z = jnp.zeros((1, y0.shape[1]), jnp.float32)
    return (y1
            + jnp.concatenate([z, y0[:-1]], axis=0)
            + jnp.concatenate([y2[1:], z], axis=0)
            + b_row)


def _ln(h, g_row, b_row):
    mu = jnp.mean(h, axis=-1, keepdims=True)
    d = h - mu
    var = jnp.mean(d * d, axis=-1, keepdims=True)
    return g_row * d * jax.lax.rsqrt(var + _EPS) + b_row


def _predict(x, mask_col, c1w_ref, c1b, g1, bt1, c2w_ref, c2b, g2, bt2,
             linw, linb):
    h = jax.nn.relu(_conv3(x, c1w_ref, c1b))
    h = _ln(h, g1, bt1)
    h = jax.nn.relu(_conv3(h, c2w_ref, c2b))
    h = _ln(h, g2, bt2)
    pred = jnp.sum(h * linw, axis=-1, keepdims=True) + linb
    return jnp.where(mask_col > 0.0, 0.0, pred)


def _onehot_lookup(v_col, lo_row, hi_row, emb_ref):
    oh = ((v_col > lo_row) & (v_col <= hi_row)).astype(jnp.float32)
    return jnp.dot(oh, emb_ref[:, :], preferred_element_type=jnp.float32)


def _va_kernel(x_ref, mask_ref, pt_ref, et_ref,
               plo_ref, phi_ref, elo_ref, ehi_ref,
               pemb_ref, eemb_ref,
               p_c1w, p_c1b, p_g1, p_bt1, p_c2w, p_c2b, p_g2, p_bt2,
               p_lw, p_lb,
               e_c1w, e_c1b, e_g1, e_bt1, e_c2w, e_c2b, e_g2, e_bt2,
               e_lw, e_lb,
               ppred_ref, pembo_ref, epred_ref, eembo_ref):
    x = x_ref[0]
    mask = mask_ref[0]

    ppred_ref[0] = _predict(x, mask, p_c1w, p_c1b[:, :], p_g1[:, :],
                            p_bt1[:, :], p_c2w, p_c2b[:, :], p_g2[:, :],
                            p_bt2[:, :], p_lw[:, :], p_lb[0, 0])
    epred_ref[0] = _predict(x, mask, e_c1w, e_c1b[:, :], e_g1[:, :],
                            e_bt1[:, :], e_c2w, e_c2b[:, :], e_g2[:, :],
                            e_bt2[:, :], e_lw[:, :], e_lb[0, 0])

    pembo_ref[0] = _onehot_lookup(pt_ref[0], plo_ref[:, :], phi_ref[:, :],
                                  pemb_ref)
    eembo_ref[0] = _onehot_lookup(et_ref[0], elo_ref[:, :], ehi_ref[:, :],
                                  eemb_ref)


def _row2(a):
    return a.reshape(1, -1)


def kernel(embeddings, src_mask, pitch_target, energy_target, pitch_bins,
           energy_bins, pitch_emb, energy_emb, p_params, e_params):
    f32 = jnp.float32
    mask_f = src_mask.astype(f32).reshape(B, T, 1)
    pt = pitch_target.reshape(B, T, 1)
    et = energy_target.reshape(B, T, 1)

    inf = jnp.full((1,), jnp.inf, f32)
    plo = jnp.concatenate([-inf, pitch_bins]).reshape(1, NBINS)
    phi = jnp.concatenate([pitch_bins, inf]).reshape(1, NBINS)
    elo = jnp.concatenate([-inf, energy_bins]).reshape(1, NBINS)
    ehi = jnp.concatenate([energy_bins, inf]).reshape(1, NBINS)

    def pp(p):
        return (p["conv1_w"], _row2(p["conv1_b"]), _row2(p["ln1_g"]),
                _row2(p["ln1_b"]), p["conv2_w"], _row2(p["conv2_b"]),
                _row2(p["ln2_g"]), _row2(p["ln2_b"]),
                p["lin_w"].reshape(1, FILT), p["lin_b"].reshape(1, 1))

    whole = lambda shape: pl.BlockSpec(shape, lambda i: (0,) * len(shape))
    per_b3 = lambda shape: pl.BlockSpec(shape, lambda i: (i, 0, 0))

    in_specs = (
        [per_b3((1, T, H)), per_b3((1, T, 1)), per_b3((1, T, 1)),
         per_b3((1, T, 1))]
        + [whole((1, NBINS))] * 4
        + [whole((NBINS, OUT))] * 2
        + [whole((K, H, FILT)), whole((1, FILT)), whole((1, FILT)),
           whole((1, FILT)), whole((K, FILT, FILT)), whole((1, FILT)),
           whole((1, FILT)), whole((1, FILT)), whole((1, FILT)),
           whole((1, 1))] * 2
    )
    out_specs = [per_b3((1, T, 1)), per_b3((1, T, OUT)),
                 per_b3((1, T, 1)), per_b3((1, T, OUT))]
    out_shape = [jax.ShapeDtypeStruct((B, T, 1), f32),
                 jax.ShapeDtypeStruct((B, T, OUT), f32),
                 jax.ShapeDtypeStruct((B, T, 1), f32),
                 jax.ShapeDtypeStruct((B, T, OUT), f32)]

    ppred, pembo, epred, eembo = pl.pallas_call(
        _va_kernel,
        grid=(B,),
        in_specs=in_specs,
        out_specs=out_specs,
        out_shape=out_shape,
    )(embeddings, mask_f, pt, et, plo, phi, elo, ehi, pitch_emb, energy_emb,
      *pp(p_params), *pp(e_params))

    return (ppred.reshape(B, T), pembo, epred.reshape(B, T), eembo)
